# Initial kernel scaffold; baseline (speedup 1.0000x reference)
import jax
import jax.numpy as jnp
from jax.experimental import pallas as pl


def kernel(observations, next_observations, actions, rewards, dones, truncations, indices):
    raise NotImplementedError("write your pallas kernel here")


# speedup vs baseline: 1.1890x; 1.1890x over previous
import functools

import jax
import jax.numpy as jnp
from jax import lax
from jax.experimental import pallas as pl
from jax.experimental.pallas import tpu as pltpu
from jax.experimental.pallas import tpu_sc as plsc

N_ENV = 512
BUF = 1024
N_OBS = 64
N_ACT = 16
BATCH = 256

B = N_ENV * BATCH
NC = 2
NS = 16
L = 16
NW = NC * NS
BW = B // NW
C = 128
NCHUNK = BW // C

_mesh = plsc.VectorSubcoreMesh(core_axis_name="c", subcore_axis_name="s")


@functools.partial(
    pl.kernel,
    mesh=_mesh,
    compiler_params=pltpu.CompilerParams(use_tc_tiling_on_sc=False),
    out_type=(
        jax.ShapeDtypeStruct((B, N_OBS), jnp.float32),
        jax.ShapeDtypeStruct((B, N_ACT), jnp.float32),
        jax.ShapeDtypeStruct((B, N_OBS), jnp.float32),
        jax.ShapeDtypeStruct((B,), jnp.float32),
        jax.ShapeDtypeStruct((B,), jnp.int32),
        jax.ShapeDtypeStruct((B,), jnp.int32),
        jax.ShapeDtypeStruct((B,), jnp.int32),
    ),
    scratch_types=[
        pltpu.VMEM((C,), jnp.int32),
        pltpu.VMEM((C,), jnp.int32),
        pltpu.VMEM((C, N_OBS), jnp.float32),
        pltpu.VMEM((C, N_OBS), jnp.float32),
        pltpu.VMEM((C, N_ACT), jnp.float32),
        pltpu.VMEM((C,), jnp.float32),
        pltpu.VMEM((C,), jnp.int32),
        pltpu.VMEM((C,), jnp.int32),
        pltpu.VMEM((C,), jnp.int32),
        pltpu.SemaphoreType.DMA,
    ],
)
def _sample(obs_h, nxt_h, act_h, rew_h, dns_h, trc_h, idx_h,
            obs_o, act_o, nxt_o, rew_o, dns_o, trc_o, ones_o,
            idx_v, gidx_v, obs_v, nxt_v, act_v, rew_v, dns_v, trc_v,
            ones_v, sem):
    wid = lax.axis_index("s") * NC + lax.axis_index("c")
    base = wid * BW

    one16 = jnp.ones((L,), jnp.int32)
    for i in range(C // L):
        ones_v[pl.ds(i * L, L)] = one16

    def chunk(k, carry):
        off = base + k * C
        pltpu.sync_copy(idx_h.at[pl.ds(off, C)], idx_v)
        for i in range(C // L):
            s0 = off + i * L
            env = s0 // BATCH
            gidx_v[pl.ds(i * L, L)] = idx_v[pl.ds(i * L, L)] + env * BUF
        cps = [
            pltpu.async_copy(obs_h.at[gidx_v], obs_v, sem),
            pltpu.async_copy(nxt_h.at[gidx_v], nxt_v, sem),
            pltpu.async_copy(act_h.at[gidx_v], act_v, sem),
            pltpu.async_copy(rew_h.at[gidx_v], rew_v, sem),
            pltpu.async_copy(dns_h.at[gidx_v], dns_v, sem),
            pltpu.async_copy(trc_h.at[gidx_v], trc_v, sem),
        ]
        for cp in cps:
            cp.wait()
        pltpu.sync_copy(obs_v, obs_o.at[pl.ds(off, C)])
        pltpu.sync_copy(act_v, act_o.at[pl.ds(off, C)])
        pltpu.sync_copy(nxt_v, nxt_o.at[pl.ds(off, C)])
        pltpu.sync_copy(rew_v, rew_o.at[pl.ds(off, C)])
        pltpu.sync_copy(dns_v, dns_o.at[pl.ds(off, C)])
        pltpu.sync_copy(trc_v, trc_o.at[pl.ds(off, C)])
        pltpu.sync_copy(ones_v, ones_o.at[pl.ds(off, C)])
        return carry

    lax.fori_loop(0, NCHUNK, chunk, 0)


def kernel(observations, next_observations, actions, rewards, dones,
           truncations, indices):
    obs_f = observations.reshape(N_ENV * BUF, N_OBS)
    nxt_f = next_observations.reshape(N_ENV * BUF, N_OBS)
    act_f = actions.reshape(N_ENV * BUF, N_ACT)
    rew_f = rewards.reshape(N_ENV * BUF)
    dns_f = dones.reshape(N_ENV * BUF)
    trc_f = truncations.reshape(N_ENV * BUF)
    idx_f = indices.reshape(B)

    obs, acts, nxt, rews, dns, trcs, ones = _sample(
        obs_f, nxt_f, act_f, rew_f, dns_f, trc_f, idx_f)
    return (obs, acts, nxt, rews, dns, trcs, ones)


# speedup vs baseline: 1.2391x; 1.0421x over previous
import functools

import jax
import jax.numpy as jnp
from jax import lax
from jax.experimental import pallas as pl
from jax.experimental.pallas import tpu as pltpu
from jax.experimental.pallas import tpu_sc as plsc

N_ENV = 512
BUF = 1024
N_OBS = 64
N_ACT = 16
BATCH = 256

B = N_ENV * BATCH
NC = 2
NS = 16
L = 16
NW = NC * NS
BW = B // NW
C = 128
NCHUNK = BW // C

_mesh = plsc.VectorSubcoreMesh(core_axis_name="c", subcore_axis_name="s")


def _data_bufs():
    return [
        pltpu.VMEM((C, N_OBS), jnp.float32),
        pltpu.VMEM((C, N_OBS), jnp.float32),
        pltpu.VMEM((C, N_ACT), jnp.float32),
        pltpu.VMEM((C,), jnp.float32),
        pltpu.VMEM((C,), jnp.int32),
        pltpu.VMEM((C,), jnp.int32),
    ]


@functools.partial(
    pl.kernel,
    mesh=_mesh,
    compiler_params=pltpu.CompilerParams(use_tc_tiling_on_sc=False),
    out_type=(
        jax.ShapeDtypeStruct((B, N_OBS), jnp.float32),
        jax.ShapeDtypeStruct((B, N_ACT), jnp.float32),
        jax.ShapeDtypeStruct((B, N_OBS), jnp.float32),
        jax.ShapeDtypeStruct((B,), jnp.float32),
        jax.ShapeDtypeStruct((B,), jnp.int32),
        jax.ShapeDtypeStruct((B,), jnp.int32),
        jax.ShapeDtypeStruct((B,), jnp.int32),
    ),
    scratch_types=(
        [pltpu.VMEM((C,), jnp.int32)] * 2 +
        [pltpu.VMEM((C,), jnp.int32)] * 2 +
        _data_bufs() + _data_bufs() +
        [pltpu.VMEM((C,), jnp.int32)] +
        [pltpu.SemaphoreType.DMA] * 4
    ),
)
def _sample(obs_h, nxt_h, act_h, rew_h, dns_h, trc_h, idx_h,
            obs_o, act_o, nxt_o, rew_o, dns_o, trc_o, ones_o,
            idx0, idx1, gidx0, gidx1,
            obs0, nxt0, act0, rew0, dns0, trc0,
            obs1, nxt1, act1, rew1, dns1, trc1,
            ones_v, gsem0, gsem1, osem0, osem1):
    wid = lax.axis_index("s") * NC + lax.axis_index("c")
    base = wid * BW

    srcs = (obs_h, nxt_h, act_h, rew_h, dns_h, trc_h)
    bufsets = (
        dict(idx=idx0, gidx=gidx0, data=(obs0, nxt0, act0, rew0, dns0, trc0),
             gsem=gsem0, osem=osem0),
        dict(idx=idx1, gidx=gidx1, data=(obs1, nxt1, act1, rew1, dns1, trc1),
             gsem=gsem1, osem=osem1),
    )
    outs = (obs_o, nxt_o, act_o, rew_o, dns_o, trc_o)

    one16 = jnp.ones((L,), jnp.int32)
    for i in range(C // L):
        ones_v[pl.ds(i * L, L)] = one16

    def stage_and_issue(bs, off):
        pltpu.sync_copy(idx_h.at[pl.ds(off, C)], bs["idx"])
        for i in range(C // L):
            s0 = off + i * L
            env = s0 // BATCH
            bs["gidx"][pl.ds(i * L, L)] = (
                bs["idx"][pl.ds(i * L, L)] + env * BUF)
        for src, dst in zip(srcs, bs["data"]):
            pltpu.make_async_copy(src.at[bs["gidx"]], dst, bs["gsem"]).start()

    def wait_gathers(bs):
        for src, dst in zip(srcs, bs["data"]):
            pltpu.make_async_copy(src.at[bs["gidx"]], dst, bs["gsem"]).wait()

    def issue_outs(bs, off):
        for dst, src in zip(outs, bs["data"]):
            pltpu.make_async_copy(src, dst.at[pl.ds(off, C)],
                                  bs["osem"]).start()
        pltpu.make_async_copy(ones_v, ones_o.at[pl.ds(off, C)],
                              bs["osem"]).start()

    def wait_outs(bs, off):
        for dst, src in zip(outs, bs["data"]):
            pltpu.make_async_copy(src, dst.at[pl.ds(off, C)],
                                  bs["osem"]).wait()
        pltpu.make_async_copy(ones_v, ones_o.at[pl.ds(off, C)],
                              bs["osem"]).wait()

    stage_and_issue(bufsets[0], base)

    def pair(j, carry):
        for b in (0, 1):
            k = 2 * j + b
            off = base + k * C
            if b == 0:
                @pl.when(j > 0)
                def _():
                    wait_outs(bufsets[1], off - C)
            else:
                wait_outs(bufsets[0], off - C)
            if b == 0:
                stage_and_issue(bufsets[1], off + C)
            else:
                @pl.when(j < NCHUNK // 2 - 1)
                def _():
                    stage_and_issue(bufsets[0], off + C)
            wait_gathers(bufsets[b])
            issue_outs(bufsets[b], off)
        return carry

    lax.fori_loop(0, NCHUNK // 2, pair, 0)
    wait_outs(bufsets[1], base + (NCHUNK - 1) * C)


def kernel(observations, next_observations, actions, rewards, dones,
           truncations, indices):
    obs_f = observations.reshape(N_ENV * BUF, N_OBS)
    nxt_f = next_observations.reshape(N_ENV * BUF, N_OBS)
    act_f = actions.reshape(N_ENV * BUF, N_ACT)
    rew_f = rewards.reshape(N_ENV * BUF)
    dns_f = dones.reshape(N_ENV * BUF)
    trc_f = truncations.reshape(N_ENV * BUF)
    idx_f = indices.reshape(B)

    obs, acts, nxt, rews, dns, trcs, ones = _sample(
        obs_f, nxt_f, act_f, rew_f, dns_f, trc_f, idx_f)
    return (obs, acts, nxt, rews, dns, trcs, ones)


# speedup vs baseline: 3.2460x; 2.6197x over previous
import functools

import jax
import jax.numpy as jnp
from jax import lax
from jax.experimental import pallas as pl
from jax.experimental.pallas import tpu as pltpu
from jax.experimental.pallas import tpu_sc as plsc

N_ENV = 512
BUF = 1024
N_OBS = 64
N_ACT = 16
BATCH = 256

B = N_ENV * BATCH
NC = 2
NS = 16
L = 16
NW = NC * NS
EPW = N_ENV // NW
GPE = BATCH // L

_mesh = plsc.VectorSubcoreMesh(core_axis_name="c", subcore_axis_name="s")


@functools.partial(
    pl.kernel,
    mesh=_mesh,
    compiler_params=pltpu.CompilerParams(use_tc_tiling_on_sc=False,
                                         needs_layout_passes=False),
    out_type=(
        jax.ShapeDtypeStruct((8, 1024, 8, 128), jnp.float32),
        jax.ShapeDtypeStruct((2, 1024, 8, 128), jnp.float32),
        jax.ShapeDtypeStruct((8, 1024, 8, 128), jnp.float32),
        jax.ShapeDtypeStruct((B,), jnp.float32),
        jax.ShapeDtypeStruct((B,), jnp.int32),
        jax.ShapeDtypeStruct((B,), jnp.int32),
        jax.ShapeDtypeStruct((B,), jnp.int32),
    ),
    scratch_types=(
        pltpu.VMEM((2 * 2048,), jnp.int32),
        pltpu.VMEM((8192,), jnp.float32),
        pltpu.VMEM((8192,), jnp.int32),
        pltpu.VMEM((8192,), jnp.int32),
        pltpu.VMEM((BATCH,), jnp.float32),
        pltpu.VMEM((BATCH,), jnp.int32),
        pltpu.VMEM((BATCH,), jnp.int32),
        pltpu.VMEM((BATCH,), jnp.int32),
        pltpu.VMEM((4 * 8192,), jnp.float32),
        pltpu.VMEM((4, 2, 8, 128), jnp.float32),
        pltpu.VMEM((2 * 8192,), jnp.float32),
        pltpu.VMEM((2, 2, 8, 128), jnp.float32),
    ),
)
def _sample(obs_h, nxt_h, act_h, rew_h, dns_h, trc_h, idx_h,
            obs_o, act_o, nxt_o, rew_o, dns_o, trc_o, ones_o,
            idx_s, rew_s, dns_s, trc_s,
            rew_v, dns_v, trc_v, ones_v,
            big_s, big_v, act_s, act_v):
    wid = lax.axis_index("s") * NC + lax.axis_index("c")
    e0 = wid * EPW
    eb0 = e0 // 8

    for j in range(GPE):
        ones_v[pl.ds(j * L, L)] = jnp.ones((L,), jnp.int32)

    pltpu.sync_copy(idx_h.at[pl.ds(eb0 * 2048, 2 * 2048)], idx_s)

    def tvec_base(ebl, er, j):
        off = ebl * 2048 + (j // 8) * 1024 + er * 128 + (j % 8) * L
        t = idx_s[pl.ds(off, L)]
        tb = jax.lax.shift_right_logical(t, 7)
        tc = jax.lax.bitwise_and(t, 127)
        return jax.lax.shift_left(tb, 10) + tc

    for ebl in range(2):
        pltpu.sync_copy(rew_h.at[pl.ds((eb0 + ebl) * 8192, 8192)], rew_s)
        pltpu.sync_copy(dns_h.at[pl.ds((eb0 + ebl) * 8192, 8192)], dns_s)
        pltpu.sync_copy(trc_h.at[pl.ds((eb0 + ebl) * 8192, 8192)], trc_s)

        def env_body(er, carry, ebl=ebl):
            def grp(j, c):
                base = tvec_base(ebl, er, j) + er * 128
                rew_v[pl.ds(j * L, L)] = plsc.load_gather(rew_s, [base])
                dns_v[pl.ds(j * L, L)] = plsc.load_gather(dns_s, [base])
                trc_v[pl.ds(j * L, L)] = plsc.load_gather(trc_s, [base])
                return c
            lax.fori_loop(0, GPE, grp, 0)
            e = (eb0 + ebl) * 8 + er
            pltpu.sync_copy(rew_v, rew_o.at[pl.ds(e * BATCH, BATCH)])
            pltpu.sync_copy(dns_v, dns_o.at[pl.ds(e * BATCH, BATCH)])
            pltpu.sync_copy(trc_v, trc_o.at[pl.ds(e * BATCH, BATCH)])
            pltpu.sync_copy(ones_v, ones_o.at[pl.ds(e * BATCH, BATCH)])
            return carry
        lax.fori_loop(0, 8, env_body, 0)

    def make_env_loop(src, dst, nfb_total, half, stage, out_v):
        nfb = min(4, nfb_total)

        def body(i, carry):
            e = e0 + i
            ebl = i // 8
            er = lax.rem(i, 8)
            pltpu.sync_copy(
                src.at[pl.ds((e * nfb_total + half * 4) * 8192,
                             nfb * 8192)], stage)

            def grp(j, c):
                base = tvec_base(ebl, er, j)
                sb = j // 8
                sc0 = (j % 8) * L
                for fb in range(nfb):
                    for fr in range(8):
                        out_v[fb, sb, fr, pl.ds(sc0, L)] = (
                            plsc.load_gather(
                                stage, [base + (fb * 8192 + fr * 128)]))
                return c
            lax.fori_loop(0, GPE, grp, 0)
            pltpu.sync_copy(
                out_v, dst.at[pl.ds(half * 4, nfb), pl.ds(2 * e, 2)])
            return carry
        return body

    for half in range(2):
        lax.fori_loop(0, EPW, make_env_loop(obs_h, obs_o, 8, half,
                                            big_s, big_v), 0)
    for half in range(2):
        lax.fori_loop(0, EPW, make_env_loop(nxt_h, nxt_o, 8, half,
                                            big_s, big_v), 0)
    lax.fori_loop(0, EPW, make_env_loop(act_h, act_o, 2, 0,
                                        act_s, act_v), 0)


def kernel(observations, next_observations, actions, rewards, dones,
           truncations, indices):
    obs5 = (observations.transpose(0, 2, 1)
            .reshape(N_ENV, 8, 8, 8, 128).transpose(0, 1, 3, 2, 4)
            .reshape(-1))
    nxt5 = (next_observations.transpose(0, 2, 1)
            .reshape(N_ENV, 8, 8, 8, 128).transpose(0, 1, 3, 2, 4)
            .reshape(-1))
    act5 = (actions.transpose(0, 2, 1)
            .reshape(N_ENV, 2, 8, 8, 128).transpose(0, 1, 3, 2, 4)
            .reshape(-1))
    rew4 = rewards.reshape(64, 8, 8, 128).transpose(0, 2, 1, 3).reshape(-1)
    dns4 = dones.reshape(64, 8, 8, 128).transpose(0, 2, 1, 3).reshape(-1)
    trc4 = truncations.reshape(64, 8, 8, 128).transpose(0, 2, 1, 3).reshape(-1)
    idx4 = indices.reshape(64, 8, 2, 128).transpose(0, 2, 1, 3).reshape(-1)

    obs_t, act_t, nxt_t, rews, dns, trcs, ones = _sample(
        obs5, nxt5, act5, rew4, dns4, trc4, idx4)

    obs = obs_t.transpose(1, 3, 0, 2).reshape(B, N_OBS)
    nxt = nxt_t.transpose(1, 3, 0, 2).reshape(B, N_OBS)
    acts = act_t.transpose(1, 3, 0, 2).reshape(B, N_ACT)
    return (obs, acts, nxt, rews, dns, trcs, ones)


# speedup vs baseline: 5.6445x; 1.7389x over previous
import functools

import jax
import jax.numpy as jnp
from jax import lax
from jax.experimental import pallas as pl
from jax.experimental.pallas import tpu as pltpu
from jax.experimental.pallas import tpu_sc as plsc

N_ENV = 512
BUF = 1024
N_OBS = 64
N_ACT = 16
BATCH = 256

B = N_ENV * BATCH
NC = 2
NS = 16
L = 16
NW = NC * NS
EPW = N_ENV // NW
GPE = BATCH // L
FBW = 8192

_mesh = plsc.VectorSubcoreMesh(core_axis_name="c", subcore_axis_name="s")


@functools.partial(
    pl.kernel,
    mesh=_mesh,
    compiler_params=pltpu.CompilerParams(use_tc_tiling_on_sc=False,
                                         needs_layout_passes=False),
    out_type=(
        jax.ShapeDtypeStruct((8, 1024, 8, 128), jnp.float32),
        jax.ShapeDtypeStruct((2, 1024, 8, 128), jnp.float32),
        jax.ShapeDtypeStruct((8, 1024, 8, 128), jnp.float32),
        jax.ShapeDtypeStruct((B,), jnp.float32),
        jax.ShapeDtypeStruct((B,), jnp.int32),
        jax.ShapeDtypeStruct((B,), jnp.int32),
        jax.ShapeDtypeStruct((B,), jnp.int32),
    ),
    scratch_types=(
        pltpu.VMEM((2 * 2048,), jnp.int32),
        pltpu.VMEM((8192,), jnp.float32),
        pltpu.VMEM((8192,), jnp.int32),
        pltpu.VMEM((8192,), jnp.int32),
        pltpu.VMEM((BATCH,), jnp.float32),
        pltpu.VMEM((BATCH,), jnp.int32),
        pltpu.VMEM((BATCH,), jnp.int32),
        pltpu.VMEM((BATCH,), jnp.int32),
        pltpu.VMEM((4 * FBW,), jnp.float32),
        pltpu.VMEM((4 * FBW,), jnp.float32),
        pltpu.VMEM((4, 2, 8, 128), jnp.float32),
        pltpu.VMEM((4, 2, 8, 128), jnp.float32),
        pltpu.SemaphoreType.DMA,
        pltpu.SemaphoreType.DMA,
        pltpu.SemaphoreType.DMA,
        pltpu.SemaphoreType.DMA,
    ),
)
def _sample(obs_h, nxt_h, act_h, rew_h, dns_h, trc_h, idx_h,
            obs_o, act_o, nxt_o, rew_o, dns_o, trc_o, ones_o,
            idx_s, rew_s, dns_s, trc_s,
            rew_v, dns_v, trc_v, ones_v,
            sbuf0, sbuf1, vbuf0, vbuf1,
            ssem0, ssem1, osem0, osem1):
    wid = lax.axis_index("s") * NC + lax.axis_index("c")
    e0 = wid * EPW
    eb0 = e0 // 8
    sbuf = (sbuf0, sbuf1)
    vbuf = (vbuf0, vbuf1)
    ssem = (ssem0, ssem1)
    osem = (osem0, osem1)

    for j in range(GPE):
        ones_v[pl.ds(j * L, L)] = jnp.ones((L,), jnp.int32)

    pltpu.sync_copy(idx_h.at[pl.ds(eb0 * 2048, 2 * 2048)], idx_s)

    def tvec_base(ebl, er, j):
        off = ebl * 2048 + (j // 8) * 1024 + er * 128 + (j % 8) * L
        t = idx_s[pl.ds(off, L)]
        tb = jax.lax.shift_right_logical(t, 7)
        tc = jax.lax.bitwise_and(t, 127)
        return jax.lax.shift_left(tb, 10) + tc

    for ebl in range(2):
        pltpu.sync_copy(rew_h.at[pl.ds((eb0 + ebl) * 8192, 8192)], rew_s)
        pltpu.sync_copy(dns_h.at[pl.ds((eb0 + ebl) * 8192, 8192)], dns_s)
        pltpu.sync_copy(trc_h.at[pl.ds((eb0 + ebl) * 8192, 8192)], trc_s)

        def env_body(er, carry, ebl=ebl):
            def grp(j, c):
                base = tvec_base(ebl, er, j) + er * 128
                rew_v[pl.ds(j * L, L)] = plsc.load_gather(rew_s, [base])
                dns_v[pl.ds(j * L, L)] = plsc.load_gather(dns_s, [base])
                trc_v[pl.ds(j * L, L)] = plsc.load_gather(trc_s, [base])
                return c
            lax.fori_loop(0, GPE, grp, 0)
            e = (eb0 + ebl) * 8 + er
            pltpu.sync_copy(rew_v, rew_o.at[pl.ds(e * BATCH, BATCH)])
            pltpu.sync_copy(dns_v, dns_o.at[pl.ds(e * BATCH, BATCH)])
            pltpu.sync_copy(trc_v, trc_o.at[pl.ds(e * BATCH, BATCH)])
            pltpu.sync_copy(ones_v, ones_o.at[pl.ds(e * BATCH, BATCH)])
            return carry
        lax.fori_loop(0, 8, env_body, 0)

    def stage_cp(src, e, fb_off, nfb_total, nfb, p):
        return pltpu.make_async_copy(
            src.at[pl.ds((e * nfb_total + fb_off) * FBW, nfb * FBW)],
            sbuf[p].at[pl.ds(0, nfb * FBW)], ssem[p])

    def out_cp(dst, e, fb_off, nfb, p):
        return pltpu.make_async_copy(
            vbuf[p].at[pl.ds(0, nfb)],
            dst.at[pl.ds(fb_off, nfb), pl.ds(2 * e, 2)], osem[p])

    def do_gather(p, ebl, er, nfb):
        def grp(j, c):
            base = tvec_base(ebl, er, j)
            sb = j // 8
            sc0 = (j % 8) * L
            for fb in range(nfb):
                for fr in range(8):
                    vbuf[p][fb, sb, fr, pl.ds(sc0, L)] = (
                        plsc.load_gather(
                            sbuf[p], [base + (fb * FBW + fr * 128)]))
            return c
        lax.fori_loop(0, GPE, grp, 0)

    def phase2(src, dst):
        stage_cp(src, e0, 0, 8, 4, 0).start()

        def body(k, carry):
            e = e0 + k
            ebl = k // 8
            er = lax.rem(k, 8)
            stage_cp(src, e, 4, 8, 4, 1).start()
            stage_cp(src, e, 0, 8, 4, 0).wait()

            @pl.when(k > 0)
            def _():
                out_cp(dst, e - 1, 0, 4, 0).wait()
            do_gather(0, ebl, er, 4)
            out_cp(dst, e, 0, 4, 0).start()

            @pl.when(k < EPW - 1)
            def _():
                stage_cp(src, e + 1, 0, 8, 4, 0).start()
            stage_cp(src, e, 4, 8, 4, 1).wait()

            @pl.when(k > 0)
            def _():
                out_cp(dst, e - 1, 4, 4, 1).wait()
            do_gather(1, ebl, er, 4)
            out_cp(dst, e, 4, 4, 1).start()
            return carry
        lax.fori_loop(0, EPW, body, 0)
        out_cp(dst, e0 + EPW - 1, 0, 4, 0).wait()
        out_cp(dst, e0 + EPW - 1, 4, 4, 1).wait()

    def phase1(src, dst):
        stage_cp(src, e0, 0, 2, 2, 0).start()

        def body(k, carry):
            iA = 2 * k
            iB = iA + 1
            eA = e0 + iA
            eB = e0 + iB
            stage_cp(src, eB, 0, 2, 2, 1).start()
            stage_cp(src, eA, 0, 2, 2, 0).wait()

            @pl.when(k > 0)
            def _():
                out_cp(dst, eA - 2, 0, 2, 0).wait()
            do_gather(0, iA // 8, lax.rem(iA, 8), 2)
            out_cp(dst, eA, 0, 2, 0).start()

            @pl.when(k < EPW // 2 - 1)
            def _():
                stage_cp(src, eA + 2, 0, 2, 2, 0).start()
            stage_cp(src, eB, 0, 2, 2, 1).wait()

            @pl.when(k > 0)
            def _():
                out_cp(dst, eB - 2, 0, 2, 1).wait()
            do_gather(1, iB // 8, lax.rem(iB, 8), 2)
            out_cp(dst, eB, 0, 2, 1).start()
            return carry
        lax.fori_loop(0, EPW // 2, body, 0)
        out_cp(dst, e0 + EPW - 2, 0, 2, 0).wait()
        out_cp(dst, e0 + EPW - 1, 0, 2, 1).wait()

    phase2(obs_h, obs_o)
    phase2(nxt_h, nxt_o)
    phase1(act_h, act_o)


def kernel(observations, next_observations, actions, rewards, dones,
           truncations, indices):
    obs5 = (observations.transpose(0, 2, 1)
            .reshape(N_ENV, 8, 8, 8, 128).transpose(0, 1, 3, 2, 4)
            .reshape(-1))
    nxt5 = (next_observations.transpose(0, 2, 1)
            .reshape(N_ENV, 8, 8, 8, 128).transpose(0, 1, 3, 2, 4)
            .reshape(-1))
    act5 = (actions.transpose(0, 2, 1)
            .reshape(N_ENV, 2, 8, 8, 128).transpose(0, 1, 3, 2, 4)
            .reshape(-1))
    rew4 = rewards.reshape(64, 8, 8, 128).transpose(0, 2, 1, 3).reshape(-1)
    dns4 = dones.reshape(64, 8, 8, 128).transpose(0, 2, 1, 3).reshape(-1)
    trc4 = truncations.reshape(64, 8, 8, 128).transpose(0, 2, 1, 3).reshape(-1)
    idx4 = indices.reshape(64, 8, 2, 128).transpose(0, 2, 1, 3).reshape(-1)

    obs_t, act_t, nxt_t, rews, dns, trcs, ones = _sample(
        obs5, nxt5, act5, rew4, dns4, trc4, idx4)

    obs = obs_t.transpose(1, 3, 0, 2).reshape(B, N_OBS)
    nxt = nxt_t.transpose(1, 3, 0, 2).reshape(B, N_OBS)
    acts = act_t.transpose(1, 3, 0, 2).reshape(B, N_ACT)
    return (obs, acts, nxt, rews, dns, trcs, ones)


# speedup vs baseline: 6.8728x; 1.2176x over previous
import functools

import jax
import jax.numpy as jnp
from jax import lax
from jax.experimental import pallas as pl
from jax.experimental.pallas import tpu as pltpu
from jax.experimental.pallas import tpu_sc as plsc

N_ENV = 512
BUF = 1024
N_OBS = 64
N_ACT = 16
BATCH = 256

B = N_ENV * BATCH
NC = 2
NS = 16
L = 16
NW = NC * NS
EPW = N_ENV // NW
GPE = BATCH // L
FBW = 8192

_mesh = plsc.VectorSubcoreMesh(core_axis_name="c", subcore_axis_name="s")


@functools.partial(
    pl.kernel,
    mesh=_mesh,
    compiler_params=pltpu.CompilerParams(use_tc_tiling_on_sc=False,
                                         needs_layout_passes=False),
    out_type=(
        jax.ShapeDtypeStruct((8, 1024, 8, 128), jnp.float32),
        jax.ShapeDtypeStruct((2, 1024, 8, 128), jnp.float32),
        jax.ShapeDtypeStruct((8, 1024, 8, 128), jnp.float32),
        jax.ShapeDtypeStruct((B,), jnp.float32),
        jax.ShapeDtypeStruct((B,), jnp.int32),
        jax.ShapeDtypeStruct((B,), jnp.int32),
        jax.ShapeDtypeStruct((B,), jnp.int32),
    ),
    scratch_types=(
        pltpu.VMEM((2 * 2048,), jnp.int32),
        pltpu.VMEM((8192,), jnp.float32),
        pltpu.VMEM((8192,), jnp.int32),
        pltpu.VMEM((8192,), jnp.int32),
        pltpu.VMEM((BATCH,), jnp.float32),
        pltpu.VMEM((BATCH,), jnp.int32),
        pltpu.VMEM((BATCH,), jnp.int32),
        pltpu.VMEM((BATCH,), jnp.int32),
        pltpu.VMEM((4 * FBW,), jnp.float32),
        pltpu.VMEM((4 * FBW,), jnp.float32),
        pltpu.VMEM((4, 2, 8, 128), jnp.float32),
        pltpu.VMEM((4, 2, 8, 128), jnp.float32),
        pltpu.SemaphoreType.DMA,
        pltpu.SemaphoreType.DMA,
        pltpu.SemaphoreType.DMA,
        pltpu.SemaphoreType.DMA,
    ),
)
def _sample(obs_h, nxt_h, act_h, rew_h, dns_h, trc_h, idx_h,
            obs_o, act_o, nxt_o, rew_o, dns_o, trc_o, ones_o,
            idx_s, rew_s, dns_s, trc_s,
            rew_v, dns_v, trc_v, ones_v,
            sbuf0, sbuf1, vbuf0, vbuf1,
            ssem0, ssem1, osem0, osem1):
    wid = lax.axis_index("s") * NC + lax.axis_index("c")
    e0 = wid * EPW
    eb0 = e0 // 8
    sbuf = (sbuf0, sbuf1)
    vbuf = (vbuf0, vbuf1)
    ssem = (ssem0, ssem1)
    osem = (osem0, osem1)

    for j in range(GPE):
        ones_v[pl.ds(j * L, L)] = jnp.ones((L,), jnp.int32)

    pltpu.sync_copy(idx_h.at[pl.ds(eb0 * 2048, 2 * 2048)], idx_s)

    def tvec_base(ebl, er, j):
        off = ebl * 2048 + (j // 8) * 1024 + er * 128 + (j % 8) * L
        t = idx_s[pl.ds(off, L)]
        tb = jax.lax.shift_right_logical(t, 7)
        tc = jax.lax.bitwise_and(t, 127)
        return jax.lax.shift_left(tb, 10) + tc

    for ebl in range(2):
        pltpu.sync_copy(rew_h.at[pl.ds((eb0 + ebl) * 8192, 8192)], rew_s)
        pltpu.sync_copy(dns_h.at[pl.ds((eb0 + ebl) * 8192, 8192)], dns_s)
        pltpu.sync_copy(trc_h.at[pl.ds((eb0 + ebl) * 8192, 8192)], trc_s)

        def env_body(er, carry, ebl=ebl):
            def grp(j, c):
                base = tvec_base(ebl, er, j) + er * 128
                rew_v[pl.ds(j * L, L)] = plsc.load_gather(rew_s, [base])
                dns_v[pl.ds(j * L, L)] = plsc.load_gather(dns_s, [base])
                trc_v[pl.ds(j * L, L)] = plsc.load_gather(trc_s, [base])
                return c
            lax.fori_loop(0, GPE, grp, 0)
            e = (eb0 + ebl) * 8 + er
            pltpu.sync_copy(rew_v, rew_o.at[pl.ds(e * BATCH, BATCH)])
            pltpu.sync_copy(dns_v, dns_o.at[pl.ds(e * BATCH, BATCH)])
            pltpu.sync_copy(trc_v, trc_o.at[pl.ds(e * BATCH, BATCH)])
            pltpu.sync_copy(ones_v, ones_o.at[pl.ds(e * BATCH, BATCH)])
            return carry
        lax.fori_loop(0, 8, env_body, 0)

    def stage_cp(src, e, fb_off, nfb_total, nfb, p):
        return pltpu.make_async_copy(
            src.at[pl.ds((e * nfb_total + fb_off) * FBW, nfb * FBW)],
            sbuf[p].at[pl.ds(0, nfb * FBW)], ssem[p])

    def out_cp(dst, e, fb_off, nfb, p):
        return pltpu.make_async_copy(
            vbuf[p].at[pl.ds(0, nfb)],
            dst.at[pl.ds(fb_off, nfb), pl.ds(2 * e, 2)], osem[p])

    def do_gather(p, ebl, er, nfb):
        @plsc.parallel_loop(0, GPE, 1, unroll=2)
        def grp(j):
            base = tvec_base(ebl, er, j)
            sb = j // 8
            sc0 = (j % 8) * L
            idx_fr = [base + fr * 128 for fr in range(8)]
            for fb in range(nfb):
                sub = sbuf[p].at[pl.ds(fb * FBW, FBW)]
                for fr in range(8):
                    vbuf[p][fb, sb, fr, pl.ds(sc0, L)] = (
                        plsc.load_gather(sub, [idx_fr[fr]]))

    def phase2(src, dst):
        stage_cp(src, e0, 0, 8, 4, 0).start()

        def body(k, carry):
            e = e0 + k
            ebl = k // 8
            er = lax.rem(k, 8)
            stage_cp(src, e, 4, 8, 4, 1).start()
            stage_cp(src, e, 0, 8, 4, 0).wait()

            @pl.when(k > 0)
            def _():
                out_cp(dst, e - 1, 0, 4, 0).wait()
            do_gather(0, ebl, er, 4)
            out_cp(dst, e, 0, 4, 0).start()

            @pl.when(k < EPW - 1)
            def _():
                stage_cp(src, e + 1, 0, 8, 4, 0).start()
            stage_cp(src, e, 4, 8, 4, 1).wait()

            @pl.when(k > 0)
            def _():
                out_cp(dst, e - 1, 4, 4, 1).wait()
            do_gather(1, ebl, er, 4)
            out_cp(dst, e, 4, 4, 1).start()
            return carry
        lax.fori_loop(0, EPW, body, 0)
        out_cp(dst, e0 + EPW - 1, 0, 4, 0).wait()
        out_cp(dst, e0 + EPW - 1, 4, 4, 1).wait()

    def phase1(src, dst):
        stage_cp(src, e0, 0, 2, 2, 0).start()

        def body(k, carry):
            iA = 2 * k
            iB = iA + 1
            eA = e0 + iA
            eB = e0 + iB
            stage_cp(src, eB, 0, 2, 2, 1).start()
            stage_cp(src, eA, 0, 2, 2, 0).wait()

            @pl.when(k > 0)
            def _():
                out_cp(dst, eA - 2, 0, 2, 0).wait()
            do_gather(0, iA // 8, lax.rem(iA, 8), 2)
            out_cp(dst, eA, 0, 2, 0).start()

            @pl.when(k < EPW // 2 - 1)
            def _():
                stage_cp(src, eA + 2, 0, 2, 2, 0).start()
            stage_cp(src, eB, 0, 2, 2, 1).wait()

            @pl.when(k > 0)
            def _():
                out_cp(dst, eB - 2, 0, 2, 1).wait()
            do_gather(1, iB // 8, lax.rem(iB, 8), 2)
            out_cp(dst, eB, 0, 2, 1).start()
            return carry
        lax.fori_loop(0, EPW // 2, body, 0)
        out_cp(dst, e0 + EPW - 2, 0, 2, 0).wait()
        out_cp(dst, e0 + EPW - 1, 0, 2, 1).wait()

    phase2(obs_h, obs_o)
    phase2(nxt_h, nxt_o)
    phase1(act_h, act_o)


def kernel(observations, next_observations, actions, rewards, dones,
           truncations, indices):
    obs5 = (observations.transpose(0, 2, 1)
            .reshape(N_ENV, 8, 8, 8, 128).transpose(0, 1, 3, 2, 4)
            .reshape(-1))
    nxt5 = (next_observations.transpose(0, 2, 1)
            .reshape(N_ENV, 8, 8, 8, 128).transpose(0, 1, 3, 2, 4)
            .reshape(-1))
    act5 = (actions.transpose(0, 2, 1)
            .reshape(N_ENV, 2, 8, 8, 128).transpose(0, 1, 3, 2, 4)
            .reshape(-1))
    rew4 = rewards.reshape(64, 8, 8, 128).transpose(0, 2, 1, 3).reshape(-1)
    dns4 = dones.reshape(64, 8, 8, 128).transpose(0, 2, 1, 3).reshape(-1)
    trc4 = truncations.reshape(64, 8, 8, 128).transpose(0, 2, 1, 3).reshape(-1)
    idx4 = indices.reshape(64, 8, 2, 128).transpose(0, 2, 1, 3).reshape(-1)

    obs_t, act_t, nxt_t, rews, dns, trcs, ones = _sample(
        obs5, nxt5, act5, rew4, dns4, trc4, idx4)

    obs = obs_t.transpose(1, 3, 0, 2).reshape(B, N_OBS)
    nxt = nxt_t.transpose(1, 3, 0, 2).reshape(B, N_OBS)
    acts = act_t.transpose(1, 3, 0, 2).reshape(B, N_ACT)
    return (obs, acts, nxt, rews, dns, trcs, ones)
